# Initial kernel scaffold; baseline (speedup 1.0000x reference)
import jax
import jax.numpy as jnp
from jax.experimental import pallas as pl


def kernel(x, peso):
    raise NotImplementedError("write your pallas kernel here")


# speedup vs baseline: 1.7037x; 1.7037x over previous
import jax
import jax.numpy as jnp
from jax import lax
from jax.experimental import pallas as pl
from jax.experimental.pallas import tpu as pltpu

_TOKENS = 32768
_DIM = 2048
_NE = 64
_K = 8
_BT = 1024


def _gate_body(x_ref, w_ref, pesos_ref, idx_ref):
    xb = x_ref[...]
    wb = w_ref[...]
    logits = lax.dot_general(
        xb, wb, (((1,), (1,)), ((), ())),
        preferred_element_type=jnp.float32,
    )
    m = jnp.max(logits, axis=1, keepdims=True)
    e = jnp.exp(logits - m)
    s = e / jnp.sum(e, axis=1, keepdims=True)

    iota = lax.broadcasted_iota(jnp.int32, (s.shape[0], _NE), 1)
    cur = s
    vals, idxs = [], []
    for _ in range(_K):
        mk = jnp.max(cur, axis=1, keepdims=True)
        hit = cur == mk
        ik = jnp.min(jnp.where(hit, iota, _NE), axis=1, keepdims=True)
        vals.append(mk)
        idxs.append(ik)
        cur = jnp.where(iota == ik, -jnp.inf, cur)
    pesos_ref[...] = jnp.concatenate(vals, axis=1)
    idx_ref[...] = jnp.concatenate(idxs, axis=1)


def kernel(x, peso):
    grid = (_TOKENS // _BT,)
    pesos, indices = pl.pallas_call(
        _gate_body,
        grid=grid,
        in_specs=[
            pl.BlockSpec((_BT, _DIM), lambda i: (i, 0)),
            pl.BlockSpec((_NE, _DIM), lambda i: (0, 0)),
        ],
        out_specs=[
            pl.BlockSpec((_BT, _K), lambda i: (i, 0)),
            pl.BlockSpec((_BT, _K), lambda i: (i, 0)),
        ],
        out_shape=[
            jax.ShapeDtypeStruct((_TOKENS, _K), jnp.float32),
            jax.ShapeDtypeStruct((_TOKENS, _K), jnp.int32),
        ],
        compiler_params=pltpu.CompilerParams(
            dimension_semantics=("arbitrary",),
        ),
    )(x, peso)
    return pesos, indices


# speedup vs baseline: 1.7686x; 1.0381x over previous
import jax
import jax.numpy as jnp
from jax import lax
from jax.experimental import pallas as pl
from jax.experimental.pallas import tpu as pltpu
from jax.experimental.pallas import tpu_sc as plsc

_TOKENS = 32768
_DIM = 2048
_NE = 64
_K = 8
_BT = 1024
_NW = 32
_RPW = _TOKENS // _NW


def _score_body(x_ref, w_ref, s_ref):
    logits = lax.dot_general(
        x_ref[...], w_ref[...], (((1,), (1,)), ((), ())),
        preferred_element_type=jnp.float32,
    )
    m = jnp.max(logits, axis=1, keepdims=True)
    e = jnp.exp(logits - m)
    s_ref[...] = e / jnp.sum(e, axis=1, keepdims=True)


def _scores(x, peso):
    return pl.pallas_call(
        _score_body,
        grid=(_TOKENS // _BT,),
        in_specs=[
            pl.BlockSpec((_BT, _DIM), lambda i: (i, 0)),
            pl.BlockSpec((_NE, _DIM), lambda i: (0, 0)),
        ],
        out_specs=pl.BlockSpec((_BT, _NE), lambda i: (i, 0)),
        out_shape=jax.ShapeDtypeStruct((_TOKENS, _NE), jnp.float32),
        compiler_params=pltpu.CompilerParams(
            dimension_semantics=("arbitrary",),
        ),
    )(x, peso)


def _merge(aK, aV, bK, bV):
    brK = lax.rev(bK, (0,))
    brV = lax.rev(bV, (0,))
    m = aK >= brK
    K = jnp.where(m, aK, brK)
    V = jnp.where(m, aV, brV)
    return plsc.sort_key_val(K, V, descending=True)


def _topk_body(s_hbm, pesos_hbm, idx_hbm, s_v, p_v, i_v):
    wid = lax.axis_index("s") * 2 + lax.axis_index("c")
    base = wid * _RPW
    pltpu.sync_copy(s_hbm.at[pl.ds(base * _NE, _RPW * _NE)], s_v)

    lane = lax.iota(jnp.int32, 16)
    mask8 = lane < 8
    v0 = lane
    v1 = lane + 16
    v2 = lane + 32
    v3 = lane + 48

    @plsc.parallel_loop(0, _RPW, unroll=4)
    def _row(r):
        off = r * _NE
        k0, i0 = plsc.sort_key_val(s_v[pl.ds(off, 16)], v0, descending=True)
        k1, i1 = plsc.sort_key_val(s_v[pl.ds(off + 16, 16)], v1, descending=True)
        k2, i2 = plsc.sort_key_val(s_v[pl.ds(off + 32, 16)], v2, descending=True)
        k3, i3 = plsc.sort_key_val(s_v[pl.ds(off + 48, 16)], v3, descending=True)
        ka, ia = _merge(k0, i0, k1, i1)
        kb, ib = _merge(k2, i2, k3, i3)
        kf, jf = _merge(ka, ia, kb, ib)
        pos = r * _K + lane
        plsc.store_scatter(p_v, [pos], kf, mask=mask8)
        plsc.store_scatter(i_v, [pos], jf, mask=mask8)

    pltpu.sync_copy(p_v, pesos_hbm.at[pl.ds(base * _K, _RPW * _K)])
    pltpu.sync_copy(i_v, idx_hbm.at[pl.ds(base * _K, _RPW * _K)])


def _topk_sc(scores_flat):
    mesh = plsc.VectorSubcoreMesh(core_axis_name="c", subcore_axis_name="s")
    f = pl.kernel(
        _topk_body,
        out_type=[
            jax.ShapeDtypeStruct((_TOKENS * _K,), jnp.float32),
            jax.ShapeDtypeStruct((_TOKENS * _K,), jnp.int32),
        ],
        mesh=mesh,
        scratch_types=[
            pltpu.VMEM((_RPW * _NE,), jnp.float32),
            pltpu.VMEM((_RPW * _K,), jnp.float32),
            pltpu.VMEM((_RPW * _K,), jnp.int32),
        ],
        compiler_params=pltpu.CompilerParams(needs_layout_passes=False),
    )
    return f(scores_flat)


def kernel(x, peso):
    scores = _scores(x, peso)
    pesos, indices = _topk_sc(scores.reshape(-1))
    return pesos.reshape(_TOKENS, _K), indices.reshape(_TOKENS, _K)
